# Initial kernel scaffold; baseline (speedup 1.0000x reference)
import jax
import jax.numpy as jnp
from jax.experimental import pallas as pl


def kernel(u, Wi_f, conv_w_f, conv_b_f, dt_bias_f, A_log_f, D_f, norm_w_f, Wo_f, Wi_b, conv_w_b, conv_b_b, dt_bias_b, A_log_b, D_b, norm_w_b, Wo_b):
    raise NotImplementedError("write your pallas kernel here")


# speedup vs baseline: 31.7165x; 31.7165x over previous
import jax
import jax.numpy as jnp
from jax.experimental import pallas as pl
from jax.experimental.pallas import tpu as pltpu

_D_MODEL = 1024
_D_STATE = 128
_D_CONV = 4
_D_INNER = 2048
_HEADDIM = 64
_NHEADS = 32
_CONV_DIM = 2304
_D_IN_PROJ = 4384
_Q = 128


def _chunk_body(u_ref, wi_ref, cw_ref, cb_ref, dtb_ref, alog_ref, d_ref,
                nw_ref, wo_ref, e_ref, out_ref, s_ref, tail_ref):
    c = pl.program_id(1)
    Q = _Q

    @pl.when(c == 0)
    def _init():
        s_ref[...] = jnp.zeros_like(s_ref)
        tail_ref[...] = jnp.zeros_like(tail_ref)

    u_blk = u_ref[0]
    zx = jnp.dot(u_blk, wi_ref[0], preferred_element_type=jnp.float32)

    z = zx[:, :_D_INNER]
    xbc_raw = zx[:, _D_INNER:_D_INNER + _CONV_DIM]
    dt_raw = zx[:, _D_INNER + _CONV_DIM:]

    ext = jnp.concatenate([tail_ref[0:3, :], xbc_raw], axis=0)
    cw = cw_ref[0]
    conv = (cw[0:1, :] * ext[0:Q, :] + cw[1:2, :] * ext[1:Q + 1, :]
            + cw[2:3, :] * ext[2:Q + 2, :] + cw[3:4, :] * ext[3:Q + 3, :])
    conv = conv + cb_ref[0]
    tail_ref[0:3, :] = xbc_raw[Q - 3:Q, :]
    xbc = conv * jax.nn.sigmoid(conv)

    x = xbc[:, :_D_INNER]
    Bm = xbc[:, _D_INNER:_D_INNER + _D_STATE]
    Cm = xbc[:, _D_INNER + _D_STATE:]

    d_arg = dt_raw + dtb_ref[0]
    dm = jnp.minimum(d_arg, 20.0)
    dt = jnp.log(1.0 + jnp.exp(dm)) + jnp.maximum(d_arg - 20.0, 0.0)
    a = dt * (-jnp.exp(alog_ref[0]))

    rows = jax.lax.broadcasted_iota(jnp.int32, (Q, Q), 0)
    cols = jax.lax.broadcasted_iota(jnp.int32, (Q, Q), 1)
    mask = rows >= cols

    tri = mask.astype(jnp.bfloat16)
    a_hi = a.astype(jnp.bfloat16)
    a_lo = (a - a_hi.astype(jnp.float32)).astype(jnp.bfloat16)
    dn = (((1,), (0,)), ((), ()))
    cs = (jax.lax.dot_general(tri, a_hi, dn, preferred_element_type=jnp.float32)
          + jax.lax.dot_general(tri, a_lo, dn, preferred_element_type=jnp.float32))
    a_sum = cs[Q - 1:Q, :]

    E = e_ref[...]

    V = jnp.concatenate([dt, dt * jnp.exp(a_sum - cs), jnp.exp(cs),
                         jnp.exp(a_sum), d_ref[0]], axis=0)
    V_hi = V.astype(jnp.bfloat16)
    V_lo = (V - V_hi.astype(jnp.float32)).astype(jnp.bfloat16)
    EX = (jax.lax.dot_general(V_hi, E, dn, preferred_element_type=jnp.float32)
          + jax.lax.dot_general(V_lo, E, dn, preferred_element_type=jnp.float32))

    dtx = EX[0:Q, :] * x
    w_st = EX[Q:2 * Q, :] * x

    S = s_ref[...]
    y_inter = jnp.dot(Cm, S, preferred_element_type=jnp.float32) * EX[2 * Q:3 * Q, :]

    s_ref[...] = (EX[3 * Q:3 * Q + 1, :] * S
                  + jax.lax.dot_general(Bm, w_st, (((0,), (0,)), ((), ())),
                                        preferred_element_type=jnp.float32))

    CB = jax.lax.dot_general(Cm, Bm, (((1,), (1,)), ((), ())),
                             preferred_element_type=jnp.float32)
    csT = cs.T
    pieces = []
    for h in range(_NHEADS):
        diff = jnp.where(mask, cs[:, h:h + 1] - csT[h:h + 1, :], -1e30)
        Mh = CB * jnp.exp(diff)
        pieces.append(jnp.dot(Mh, dtx[:, h * _HEADDIM:(h + 1) * _HEADDIM],
                              preferred_element_type=jnp.float32))
    y_intra = jnp.concatenate(pieces, axis=1)

    y = y_intra + y_inter + EX[3 * Q + 1:3 * Q + 2, :] * x

    y = y * (z * jax.nn.sigmoid(z))
    ms = jnp.mean(y * y, axis=1, keepdims=True)
    y = y * jax.lax.rsqrt(ms + 1e-5) * nw_ref[0]
    out_ref[0] = jnp.dot(y.astype(jnp.bfloat16), wo_ref[0],
                         preferred_element_type=jnp.float32)


@jax.jit
def kernel(u, Wi_f, conv_w_f, conv_b_f, dt_bias_f, A_log_f, D_f, norm_w_f,
           Wo_f, Wi_b, conv_w_b, conv_b_b, dt_bias_b, A_log_b, D_b, norm_w_b,
           Wo_b):
    L = u.shape[1]
    nchunks = L // _Q

    u_all = jnp.concatenate([u, u[:, ::-1]], axis=0).astype(jnp.bfloat16)
    wi_all = jnp.stack([Wi_f.T, Wi_b.T]).astype(jnp.bfloat16)
    wo_all = jnp.stack([Wo_f.T, Wo_b.T]).astype(jnp.bfloat16)
    cw_all = jnp.stack([conv_w_f[:, 0, :].T, conv_w_b[:, 0, :].T])
    cb_all = jnp.stack([conv_b_f, conv_b_b])[:, None, :]
    dtb_all = jnp.stack([dt_bias_f, dt_bias_b])[:, None, :]
    alog_all = jnp.stack([A_log_f, A_log_b])[:, None, :]
    d_all = jnp.stack([D_f, D_b])[:, None, :]
    nw_all = jnp.stack([norm_w_f, norm_w_b])[:, None, :]

    heads = jnp.arange(_NHEADS, dtype=jnp.int32)[:, None]
    cols = jnp.arange(_D_INNER, dtype=jnp.int32)[None, :]
    E = (cols // _HEADDIM == heads).astype(jnp.bfloat16)

    wsel = lambda a, c: (a // 2, 0, 0)
    res = pl.pallas_call(
        _chunk_body,
        grid=(4, nchunks),
        in_specs=[
            pl.BlockSpec((1, _Q, _D_MODEL), lambda a, c: (a, c, 0)),
            pl.BlockSpec((1, _D_MODEL, _D_IN_PROJ), wsel),
            pl.BlockSpec((1, _D_CONV, _CONV_DIM), wsel),
            pl.BlockSpec((1, 1, _CONV_DIM), wsel),
            pl.BlockSpec((1, 1, _NHEADS), wsel),
            pl.BlockSpec((1, 1, _NHEADS), wsel),
            pl.BlockSpec((1, 1, _NHEADS), wsel),
            pl.BlockSpec((1, 1, _D_INNER), wsel),
            pl.BlockSpec((1, _D_INNER, _D_MODEL), wsel),
            pl.BlockSpec((_NHEADS, _D_INNER), lambda a, c: (0, 0)),
        ],
        out_specs=pl.BlockSpec((1, _Q, _D_MODEL), lambda a, c: (a, c, 0)),
        out_shape=jax.ShapeDtypeStruct((4, L, _D_MODEL), jnp.float32),
        scratch_shapes=[
            pltpu.VMEM((_D_STATE, _D_INNER), jnp.float32),
            pltpu.VMEM((8, _CONV_DIM), jnp.float32),
        ],
        compiler_params=pltpu.CompilerParams(
            dimension_semantics=("parallel", "arbitrary"),
            vmem_limit_bytes=56 * 1024 * 1024,
        ),
    )(u_all, wi_all, cw_all, cb_all, dtb_all, alog_all, d_all, nw_all,
      wo_all, E)

    res = res.reshape(2, 2, L, _D_MODEL)
    return (res[0] + res[1][:, ::-1]) * 0.5


# speedup vs baseline: 39.8729x; 1.2572x over previous
import jax
import jax.numpy as jnp
from jax.experimental import pallas as pl
from jax.experimental.pallas import tpu as pltpu

_D_MODEL = 1024
_D_STATE = 128
_D_CONV = 4
_D_INNER = 2048
_HEADDIM = 64
_NHEADS = 32
_CONV_DIM = 2304
_D_IN_PROJ = 4384
_Q = 128


def _chunk_body(u_ref, wi_ref, cw_ref, cb_ref, dtb_ref, alog_ref, d_ref,
                nw_ref, wo_ref, e_ref, out_ref, s_ref, tail_ref):
    c = pl.program_id(1)
    Q = _Q
    is_rev = pl.program_id(0) >= 2

    @pl.when(c == 0)
    def _init():
        s_ref[...] = jnp.zeros_like(s_ref)
        tail_ref[...] = jnp.zeros_like(tail_ref)

    rows_q = jax.lax.broadcasted_iota(jnp.int32, (Q, Q), 0)
    cols_q = jax.lax.broadcasted_iota(jnp.int32, (Q, Q), 1)
    R16 = (rows_q + cols_q == Q - 1).astype(jnp.bfloat16)
    u_blk = u_ref[0]
    u_flip = jnp.dot(R16, u_blk, preferred_element_type=jnp.float32
                     ).astype(jnp.bfloat16)
    u_blk = jnp.where(is_rev, u_flip, u_blk)
    zx = jax.lax.dot_general(u_blk, wi_ref[0], (((1,), (1,)), ((), ())),
                             preferred_element_type=jnp.float32)

    z = zx[:, :_D_INNER]
    xbc_raw = zx[:, _D_INNER:_D_INNER + _CONV_DIM]
    dt_raw = zx[:, _D_INNER + _CONV_DIM:]

    ext = jnp.concatenate([tail_ref[0:3, :], xbc_raw], axis=0)
    cw = cw_ref[0]
    conv = (cw[0:1, :] * ext[0:Q, :] + cw[1:2, :] * ext[1:Q + 1, :]
            + cw[2:3, :] * ext[2:Q + 2, :] + cw[3:4, :] * ext[3:Q + 3, :])
    conv = conv + cb_ref[0]
    tail_ref[0:3, :] = xbc_raw[Q - 3:Q, :]
    xbc = conv * jax.nn.sigmoid(conv)

    x = xbc[:, :_D_INNER]
    Bm = xbc[:, _D_INNER:_D_INNER + _D_STATE]
    Cm = xbc[:, _D_INNER + _D_STATE:]

    d_arg = dt_raw + dtb_ref[0]
    dm = jnp.minimum(d_arg, 20.0)
    dt = jnp.log(1.0 + jnp.exp(dm)) + jnp.maximum(d_arg - 20.0, 0.0)
    a = dt * (-jnp.exp(alog_ref[0]))

    mask = rows_q >= cols_q

    tri = mask.astype(jnp.bfloat16)
    a_hi = a.astype(jnp.bfloat16)
    a_lo = (a - a_hi.astype(jnp.float32)).astype(jnp.bfloat16)
    dn = (((1,), (0,)), ((), ()))
    cs = (jax.lax.dot_general(tri, a_hi, dn, preferred_element_type=jnp.float32)
          + jax.lax.dot_general(tri, a_lo, dn, preferred_element_type=jnp.float32))
    a_sum = cs[Q - 1:Q, :]

    E = e_ref[...]

    V = jnp.concatenate([dt, dt * jnp.exp(a_sum - cs), jnp.exp(cs),
                         jnp.exp(a_sum), d_ref[0]], axis=0)
    V_hi = V.astype(jnp.bfloat16)
    V_lo = (V - V_hi.astype(jnp.float32)).astype(jnp.bfloat16)
    EX = (jax.lax.dot_general(V_hi, E, dn, preferred_element_type=jnp.float32)
          + jax.lax.dot_general(V_lo, E, dn, preferred_element_type=jnp.float32))

    dtx = EX[0:Q, :] * x
    w_st = EX[Q:2 * Q, :] * x

    S = s_ref[...]
    y_inter = jnp.dot(Cm, S, preferred_element_type=jnp.float32) * EX[2 * Q:3 * Q, :]

    s_ref[...] = (EX[3 * Q:3 * Q + 1, :] * S
                  + jax.lax.dot_general(Bm, w_st, (((0,), (0,)), ((), ())),
                                        preferred_element_type=jnp.float32))

    CB = jax.lax.dot_general(Cm, Bm, (((1,), (1,)), ((), ())),
                             preferred_element_type=jnp.float32)
    csT = cs.T
    pieces = []
    for h in range(_NHEADS):
        diff = jnp.where(mask, cs[:, h:h + 1] - csT[h:h + 1, :], -1e30)
        Mh = CB * jnp.exp(diff)
        pieces.append(jnp.dot(Mh, dtx[:, h * _HEADDIM:(h + 1) * _HEADDIM],
                              preferred_element_type=jnp.float32))
    y_intra = jnp.concatenate(pieces, axis=1)

    y = y_intra + y_inter + EX[3 * Q + 1:3 * Q + 2, :] * x

    y = y * (z * jax.nn.sigmoid(z))
    ms = jnp.mean(y * y, axis=1, keepdims=True)
    y = y * jax.lax.rsqrt(ms + 1e-5) * nw_ref[0]
    y16 = y.astype(jnp.bfloat16)
    y_flip = jnp.dot(R16, y16, preferred_element_type=jnp.float32
                     ).astype(jnp.bfloat16)
    y16 = jnp.where(is_rev, y_flip, y16)
    out_ref[0] = jax.lax.dot_general(y16, wo_ref[0], (((1,), (1,)), ((), ())),
                                     preferred_element_type=jnp.float32)


@jax.jit
def kernel(u, Wi_f, conv_w_f, conv_b_f, dt_bias_f, A_log_f, D_f, norm_w_f,
           Wo_f, Wi_b, conv_w_b, conv_b_b, dt_bias_b, A_log_b, D_b, norm_w_b,
           Wo_b):
    L = u.shape[1]
    nchunks = L // _Q

    u_all = u.astype(jnp.bfloat16)
    wi_all = jnp.stack([Wi_f, Wi_b]).astype(jnp.bfloat16)
    wo_all = jnp.stack([Wo_f, Wo_b]).astype(jnp.bfloat16)
    cw_all = jnp.stack([conv_w_f[:, 0, :].T, conv_w_b[:, 0, :].T])
    cb_all = jnp.stack([conv_b_f, conv_b_b])[:, None, :]
    dtb_all = jnp.stack([dt_bias_f, dt_bias_b])[:, None, :]
    alog_all = jnp.stack([A_log_f, A_log_b])[:, None, :]
    d_all = jnp.stack([D_f, D_b])[:, None, :]
    nw_all = jnp.stack([norm_w_f, norm_w_b])[:, None, :]

    heads = jnp.arange(_NHEADS, dtype=jnp.int32)[:, None]
    cols = jnp.arange(_D_INNER, dtype=jnp.int32)[None, :]
    E = (cols // _HEADDIM == heads).astype(jnp.bfloat16)

    wsel = lambda a, c: (a // 2, 0, 0)
    tsel = lambda a, c: (a % 2, c + (a // 2) * (nchunks - 1 - 2 * c), 0)
    res = pl.pallas_call(
        _chunk_body,
        grid=(4, nchunks),
        in_specs=[
            pl.BlockSpec((1, _Q, _D_MODEL), tsel),
            pl.BlockSpec((1, _D_IN_PROJ, _D_MODEL), wsel),
            pl.BlockSpec((1, _D_CONV, _CONV_DIM), wsel),
            pl.BlockSpec((1, 1, _CONV_DIM), wsel),
            pl.BlockSpec((1, 1, _NHEADS), wsel),
            pl.BlockSpec((1, 1, _NHEADS), wsel),
            pl.BlockSpec((1, 1, _NHEADS), wsel),
            pl.BlockSpec((1, 1, _D_INNER), wsel),
            pl.BlockSpec((1, _D_MODEL, _D_INNER), wsel),
            pl.BlockSpec((_NHEADS, _D_INNER), lambda a, c: (0, 0)),
        ],
        out_specs=pl.BlockSpec((1, _Q, _D_MODEL),
                               lambda a, c: (a, c + (a // 2) * (nchunks - 1 - 2 * c), 0)),
        out_shape=jax.ShapeDtypeStruct((4, L, _D_MODEL), jnp.float32),
        scratch_shapes=[
            pltpu.VMEM((_D_STATE, _D_INNER), jnp.float32),
            pltpu.VMEM((8, _CONV_DIM), jnp.float32),
        ],
        compiler_params=pltpu.CompilerParams(
            dimension_semantics=("parallel", "arbitrary"),
            vmem_limit_bytes=56 * 1024 * 1024,
        ),
    )(u_all, wi_all, cw_all, cb_all, dtb_all, alog_all, d_all, nw_all,
      wo_all, E)

    return (res[0:2] + res[2:4]) * 0.5


# speedup vs baseline: 53.4642x; 1.3409x over previous
import jax
import jax.numpy as jnp
from jax.experimental import pallas as pl
from jax.experimental.pallas import tpu as pltpu

_D_MODEL = 1024
_D_STATE = 128
_D_CONV = 4
_D_INNER = 2048
_HEADDIM = 64
_NHEADS = 32
_CONV_DIM = 2304
_D_IN_PROJ = 4384
_Q = 128


def _chunk_body(u_ref, wi_ref, cw_ref, cb_ref, dtb_ref, alog_ref, d_ref,
                nw_ref, wo_ref, e_ref, out_ref, s_ref, tail_ref,
                wi_t_ref, wo_t_ref):
    c = pl.program_id(1)
    Q = _Q
    is_rev = pl.program_id(0) >= 2

    @pl.when(c == 0)
    def _init():
        s_ref[...] = jnp.zeros_like(s_ref)
        tail_ref[...] = jnp.zeros_like(tail_ref)
        wi_t_ref[...] = wi_ref[0].T
        wo_t_ref[...] = wo_ref[0].T

    rows_q = jax.lax.broadcasted_iota(jnp.int32, (Q, Q), 0)
    cols_q = jax.lax.broadcasted_iota(jnp.int32, (Q, Q), 1)
    R16 = (rows_q + cols_q == Q - 1).astype(jnp.bfloat16)
    u_blk = u_ref[0]
    u_flip = jnp.dot(R16, u_blk, preferred_element_type=jnp.float32
                     ).astype(jnp.bfloat16)
    u_blk = jnp.where(is_rev, u_flip, u_blk)
    zx = jnp.dot(u_blk, wi_t_ref[...], preferred_element_type=jnp.float32)

    z = zx[:, :_D_INNER]
    xbc_raw = zx[:, _D_INNER:_D_INNER + _CONV_DIM]
    dt_raw = zx[:, _D_INNER + _CONV_DIM:]

    ext = jnp.concatenate([tail_ref[0:3, :], xbc_raw], axis=0)
    cw = cw_ref[0]
    conv = (cw[0:1, :] * ext[0:Q, :] + cw[1:2, :] * ext[1:Q + 1, :]
            + cw[2:3, :] * ext[2:Q + 2, :] + cw[3:4, :] * ext[3:Q + 3, :])
    conv = conv + cb_ref[0]
    tail_ref[0:3, :] = xbc_raw[Q - 3:Q, :]
    xbc = conv * jax.nn.sigmoid(conv)

    x = xbc[:, :_D_INNER]
    Bm = xbc[:, _D_INNER:_D_INNER + _D_STATE]
    Cm = xbc[:, _D_INNER + _D_STATE:]

    d_arg = dt_raw + dtb_ref[0]
    dm = jnp.minimum(d_arg, 20.0)
    dt = jnp.log(1.0 + jnp.exp(dm)) + jnp.maximum(d_arg - 20.0, 0.0)
    a = dt * (-jnp.exp(alog_ref[0]))

    mask = rows_q >= cols_q

    tri = mask.astype(jnp.bfloat16)
    a_hi = a.astype(jnp.bfloat16)
    a_lo = (a - a_hi.astype(jnp.float32)).astype(jnp.bfloat16)
    dn = (((1,), (0,)), ((), ()))
    cs = (jax.lax.dot_general(tri, a_hi, dn, preferred_element_type=jnp.float32)
          + jax.lax.dot_general(tri, a_lo, dn, preferred_element_type=jnp.float32))
    a_sum = cs[Q - 1:Q, :]

    E = e_ref[...]

    V = jnp.concatenate([dt, dt * jnp.exp(a_sum - cs), jnp.exp(cs),
                         jnp.exp(a_sum), d_ref[0]], axis=0)
    V_hi = V.astype(jnp.bfloat16)
    V_lo = (V - V_hi.astype(jnp.float32)).astype(jnp.bfloat16)
    EX = (jax.lax.dot_general(V_hi, E, dn, preferred_element_type=jnp.float32)
          + jax.lax.dot_general(V_lo, E, dn, preferred_element_type=jnp.float32))

    dtx = EX[0:Q, :] * x
    w_st = EX[Q:2 * Q, :] * x

    S = s_ref[...]
    y_inter = jnp.dot(Cm, S, preferred_element_type=jnp.float32) * EX[2 * Q:3 * Q, :]

    s_ref[...] = (EX[3 * Q:3 * Q + 1, :] * S
                  + jax.lax.dot_general(Bm, w_st, (((0,), (0,)), ((), ())),
                                        preferred_element_type=jnp.float32))

    CB = jax.lax.dot_general(Cm, Bm, (((1,), (1,)), ((), ())),
                             preferred_element_type=jnp.float32)
    csT = cs.T
    pieces = []
    for h in range(_NHEADS):
        diff = jnp.where(mask, cs[:, h:h + 1] - csT[h:h + 1, :], -1e30)
        Mh = CB * jnp.exp(diff)
        pieces.append(jnp.dot(Mh, dtx[:, h * _HEADDIM:(h + 1) * _HEADDIM],
                              preferred_element_type=jnp.float32))
    y_intra = jnp.concatenate(pieces, axis=1)

    y = y_intra + y_inter + EX[3 * Q + 1:3 * Q + 2, :] * x

    y = y * (z * jax.nn.sigmoid(z))
    ms = jnp.mean(y * y, axis=1, keepdims=True)
    y = y * jax.lax.rsqrt(ms + 1e-5) * nw_ref[0]
    y16 = y.astype(jnp.bfloat16)
    y_flip = jnp.dot(R16, y16, preferred_element_type=jnp.float32
                     ).astype(jnp.bfloat16)
    y16 = jnp.where(is_rev, y_flip, y16)
    out_ref[0] = jnp.dot(y16, wo_t_ref[...], preferred_element_type=jnp.float32)


@jax.jit
def kernel(u, Wi_f, conv_w_f, conv_b_f, dt_bias_f, A_log_f, D_f, norm_w_f,
           Wo_f, Wi_b, conv_w_b, conv_b_b, dt_bias_b, A_log_b, D_b, norm_w_b,
           Wo_b):
    L = u.shape[1]
    nchunks = L // _Q

    u_all = u.astype(jnp.bfloat16)
    wi_all = jnp.stack([Wi_f, Wi_b]).astype(jnp.bfloat16)
    wo_all = jnp.stack([Wo_f, Wo_b]).astype(jnp.bfloat16)
    cw_all = jnp.stack([conv_w_f[:, 0, :].T, conv_w_b[:, 0, :].T])
    cb_all = jnp.stack([conv_b_f, conv_b_b])[:, None, :]
    dtb_all = jnp.stack([dt_bias_f, dt_bias_b])[:, None, :]
    alog_all = jnp.stack([A_log_f, A_log_b])[:, None, :]
    d_all = jnp.stack([D_f, D_b])[:, None, :]
    nw_all = jnp.stack([norm_w_f, norm_w_b])[:, None, :]

    heads = jnp.arange(_NHEADS, dtype=jnp.int32)[:, None]
    cols = jnp.arange(_D_INNER, dtype=jnp.int32)[None, :]
    E = (cols // _HEADDIM == heads).astype(jnp.bfloat16)

    wsel = lambda a, c: (a // 2, 0, 0)
    tsel = lambda a, c: (a % 2, c + (a // 2) * (nchunks - 1 - 2 * c), 0)
    res = pl.pallas_call(
        _chunk_body,
        grid=(4, nchunks),
        in_specs=[
            pl.BlockSpec((1, _Q, _D_MODEL), tsel),
            pl.BlockSpec((1, _D_IN_PROJ, _D_MODEL), wsel),
            pl.BlockSpec((1, _D_CONV, _CONV_DIM), wsel),
            pl.BlockSpec((1, 1, _CONV_DIM), wsel),
            pl.BlockSpec((1, 1, _NHEADS), wsel),
            pl.BlockSpec((1, 1, _NHEADS), wsel),
            pl.BlockSpec((1, 1, _NHEADS), wsel),
            pl.BlockSpec((1, 1, _D_INNER), wsel),
            pl.BlockSpec((1, _D_MODEL, _D_INNER), wsel),
            pl.BlockSpec((_NHEADS, _D_INNER), lambda a, c: (0, 0)),
        ],
        out_specs=pl.BlockSpec((1, _Q, _D_MODEL),
                               lambda a, c: (a, c + (a // 2) * (nchunks - 1 - 2 * c), 0)),
        out_shape=jax.ShapeDtypeStruct((4, L, _D_MODEL), jnp.float32),
        scratch_shapes=[
            pltpu.VMEM((_D_STATE, _D_INNER), jnp.float32),
            pltpu.VMEM((8, _CONV_DIM), jnp.float32),
            pltpu.VMEM((_D_MODEL, _D_IN_PROJ), jnp.bfloat16),
            pltpu.VMEM((_D_INNER, _D_MODEL), jnp.bfloat16),
        ],
        compiler_params=pltpu.CompilerParams(
            dimension_semantics=("parallel", "arbitrary"),
            vmem_limit_bytes=56 * 1024 * 1024,
        ),
    )(u_all, wi_all, cw_all, cb_all, dtb_all, alog_all, d_all, nw_all,
      wo_all, E)

    return (res[0:2] + res[2:4]) * 0.5
